# baseline (speedup 1.0000x reference)
import math

import jax
import jax.numpy as jnp
from jax.experimental import pallas as pl
from jax.experimental.pallas import tpu as pltpu

B, N, F, H, T = 8, 512, 128, 128, 16
MASKED = -1.0
_INV_SQRT_H = 1.0 / math.sqrt(H)


def _edge_kernel(ns_ref, skip_ref, x_ref, adj_ref, a_ref,
                 we2h, be2h, wm_pr, wu_pr,
                 wm_bf, wu_bf, we12,
                 out_ref, zc_ref):
    g = pl.program_id(0)
    f32 = jnp.float32
    LANES = 4
    ns = [ns_ref[LANES * g + l] for l in range(LANES)]

    def net_step(l, z, h, wm, wu):
        m = jnp.maximum(jnp.dot(z + h, wm[...],
                                preferred_element_type=f32), 0.0)
        msg = jnp.dot(adj_ref[l], m, preferred_element_type=f32)
        zc_ref[l, :, H:2 * H] = msg
        return jnp.maximum(jnp.dot(zc_ref[l], wu[...],
                                   preferred_element_type=f32), 0.0)

    z2 = [jnp.tanh(jnp.dot(x_ref[l], we2h[...], preferred_element_type=f32)
                   + be2h[...]) for l in range(LANES)]
    z_pr = [z[:, 0:H] for z in z2]
    z_bf = [z[:, H:2 * H] for z in z2]

    for l in range(LANES):
        zc_ref[l, :, 0:H] = z_pr[l]
    zero = jnp.zeros((N, H), f32)

    hfin = [None] * LANES
    for a, bl in ((0, 1), (2, 3)):
        nmin = jnp.minimum(ns[a], ns[bl])
        hs = jax.lax.fori_loop(
            0, nmin,
            lambda i, hh: (net_step(a, z_pr[a], hh[0], wm_pr, wu_pr),
                           net_step(bl, z_pr[bl], hh[1], wm_pr, wu_pr)),
            (zero, zero))
        hfin[a] = jax.lax.fori_loop(
            nmin, ns[a],
            lambda i, hh: net_step(a, z_pr[a], hh, wm_pr, wu_pr), hs[0])
        hfin[bl] = jax.lax.fori_loop(
            nmin, ns[bl],
            lambda i, hh: net_step(bl, z_pr[bl], hh, wm_pr, wu_pr), hs[1])

    for l in range(LANES):
        zc_ref[l, :, 0:H] = z_bf[l]
    for l in range(LANES):
        hb = net_step(l, z_bf[l], hfin[l], wm_bf, wu_bf)
        e12 = jnp.dot(hb, we12[...], preferred_element_type=f32)
        cand = jax.lax.dot_general(
            e12[:, 0:H], e12[:, H:2 * H], (((1,), (1,)), ((), ())),
            preferred_element_type=f32) * _INV_SQRT_H
        out_ref[l] = jnp.where(skip_ref[LANES * g + l] != 0, MASKED,
                               a_ref[l] * cand)


def kernel(x, adj, A, W_enc_pr, b_enc_pr, W_msg_pr, W_upd_pr, w_node_pr,
           We1_pr, We2_pr, W_enc_bfs, b_enc_bfs, W_msg_bfs, W_upd_bfs,
           w_node_bfs, We1_bfs, We2_bfs, phase_logits, lengths):
    del w_node_pr, We1_pr, We2_pr, w_node_bfs

    p = jnp.argmax(phase_logits, axis=-1).astype(jnp.int32)
    iv = jnp.arange(T, dtype=jnp.int32)[:, None]
    ln = lengths.astype(jnp.int32)[None, :]
    valid = (iv >= 1) & (ln > iv + 1) & (p == 0)
    i_star = jnp.max(jnp.where(valid, iv, 0), axis=0)
    reset = (p == 1) & (iv < i_star[None, :])
    j0 = jnp.max(jnp.where(reset, iv + 1, 0), axis=0)
    nsteps = i_star - j0 + 1
    skip = ((i_star == 0) & (p[0] == 1)).astype(jnp.int32)

    we2h = jnp.concatenate([W_enc_pr, W_enc_bfs], axis=1)
    be2h = jnp.concatenate([b_enc_pr, b_enc_bfs]).reshape(1, 2 * H)
    we12 = jnp.concatenate([We1_bfs, We2_bfs], axis=1)

    smem = pl.BlockSpec(memory_space=pltpu.SMEM)
    full = lambda *shape: pl.BlockSpec(shape, lambda b: (0,) * len(shape))
    batched = lambda *shape: pl.BlockSpec((4,) + shape, lambda b: (b, 0, 0))

    out = pl.pallas_call(
        _edge_kernel,
        grid=(B // 4,),
        in_specs=[
            smem, smem,
            batched(N, F), batched(N, N), batched(N, N),
            full(F, 2 * H), full(1, 2 * H), full(H, H), full(2 * H, H),
            full(H, H), full(2 * H, H), full(H, 2 * H),
        ],
        out_specs=batched(N, N),
        out_shape=jax.ShapeDtypeStruct((B, N, N), jnp.float32),
        scratch_shapes=[pltpu.VMEM((4, N, 2 * H), jnp.float32)],
    )(nsteps, skip, x, adj, A,
      we2h, be2h, W_msg_pr, W_upd_pr, W_msg_bfs, W_upd_bfs, we12)
    return out


# speedup vs baseline: 1.0420x; 1.0420x over previous
import math

import jax
import jax.numpy as jnp
from jax.experimental import pallas as pl
from jax.experimental.pallas import tpu as pltpu

B, N, F, H, T = 8, 512, 128, 128, 16
MASKED = -1.0
_INV_SQRT_H = 1.0 / math.sqrt(H)


def _edge_kernel(ns_ref, skip_ref, x_ref, adj_ref, a_ref,
                 we2h, be2h, wm_pr, wu_pr,
                 wm_bf, wu_bf, we12,
                 out_ref, zc_ref):
    g = pl.program_id(0)
    f32 = jnp.float32
    ns0 = ns_ref[2 * g]
    ns1 = ns_ref[2 * g + 1]
    nmin = jnp.minimum(ns0, ns1)
    adjs = (adj_ref[0], adj_ref[1])

    def net_step(l, z, h, wm, wu):
        m = jnp.maximum(jnp.dot(z + h, wm[...],
                                preferred_element_type=f32), 0.0)
        msg = jnp.dot(adjs[l], m, preferred_element_type=f32)
        zc_ref[l, :, H:2 * H] = msg
        return jnp.maximum(jnp.dot(zc_ref[l], wu[...],
                                   preferred_element_type=f32), 0.0)

    z2 = [jnp.tanh(jnp.dot(x_ref[l], we2h[...], preferred_element_type=f32)
                   + be2h[...]) for l in (0, 1)]
    z_pr = (z2[0][:, 0:H], z2[1][:, 0:H])
    z_bf = (z2[0][:, H:2 * H], z2[1][:, H:2 * H])

    zc_ref[0, :, 0:H] = z_pr[0]
    zc_ref[1, :, 0:H] = z_pr[1]
    zero = jnp.zeros((N, H), f32)

    def joint(hh):
        return (net_step(0, z_pr[0], hh[0], wm_pr, wu_pr),
                net_step(1, z_pr[1], hh[1], wm_pr, wu_pr))

    hs = jax.lax.fori_loop(
        0, nmin // 2, lambda i, hh: joint(joint(hh)), (zero, zero))
    hs = jax.lax.fori_loop(
        2 * (nmin // 2), nmin, lambda i, hh: joint(hh), hs)

    h0 = jax.lax.fori_loop(
        nmin, ns0, lambda i, hh: net_step(0, z_pr[0], hh, wm_pr, wu_pr),
        hs[0])
    h1 = jax.lax.fori_loop(
        nmin, ns1, lambda i, hh: net_step(1, z_pr[1], hh, wm_pr, wu_pr),
        hs[1])

    zc_ref[0, :, 0:H] = z_bf[0]
    zc_ref[1, :, 0:H] = z_bf[1]
    for l, h in ((0, h0), (1, h1)):
        hb = net_step(l, z_bf[l], h, wm_bf, wu_bf)
        e12 = jnp.dot(hb, we12[...], preferred_element_type=f32)
        cand = jax.lax.dot_general(
            e12[:, 0:H], e12[:, H:2 * H], (((1,), (1,)), ((), ())),
            preferred_element_type=f32) * _INV_SQRT_H
        out_ref[l] = jnp.where(skip_ref[2 * g + l] != 0, MASKED,
                               a_ref[l] * cand)


def kernel(x, adj, A, W_enc_pr, b_enc_pr, W_msg_pr, W_upd_pr, w_node_pr,
           We1_pr, We2_pr, W_enc_bfs, b_enc_bfs, W_msg_bfs, W_upd_bfs,
           w_node_bfs, We1_bfs, We2_bfs, phase_logits, lengths):
    del w_node_pr, We1_pr, We2_pr, w_node_bfs

    p = jnp.argmax(phase_logits, axis=-1).astype(jnp.int32)
    iv = jnp.arange(T, dtype=jnp.int32)[:, None]
    ln = lengths.astype(jnp.int32)[None, :]
    valid = (iv >= 1) & (ln > iv + 1) & (p == 0)
    i_star = jnp.max(jnp.where(valid, iv, 0), axis=0)
    reset = (p == 1) & (iv < i_star[None, :])
    j0 = jnp.max(jnp.where(reset, iv + 1, 0), axis=0)
    nsteps = i_star - j0 + 1
    skip = ((i_star == 0) & (p[0] == 1)).astype(jnp.int32)

    we2h = jnp.concatenate([W_enc_pr, W_enc_bfs], axis=1)
    be2h = jnp.concatenate([b_enc_pr, b_enc_bfs]).reshape(1, 2 * H)
    we12 = jnp.concatenate([We1_bfs, We2_bfs], axis=1)

    smem = pl.BlockSpec(memory_space=pltpu.SMEM)
    full = lambda *shape: pl.BlockSpec(shape, lambda b: (0,) * len(shape))
    batched = lambda *shape: pl.BlockSpec((2,) + shape, lambda b: (b, 0, 0))

    out = pl.pallas_call(
        _edge_kernel,
        grid=(B // 2,),
        in_specs=[
            smem, smem,
            batched(N, F), batched(N, N), batched(N, N),
            full(F, 2 * H), full(1, 2 * H), full(H, H), full(2 * H, H),
            full(H, H), full(2 * H, H), full(H, 2 * H),
        ],
        out_specs=batched(N, N),
        out_shape=jax.ShapeDtypeStruct((B, N, N), jnp.float32),
        scratch_shapes=[pltpu.VMEM((2, N, 2 * H), jnp.float32)],
    )(nsteps, skip, x, adj, A,
      we2h, be2h, W_msg_pr, W_upd_pr, W_msg_bfs, W_upd_bfs, we12)
    return out


# speedup vs baseline: 1.0480x; 1.0058x over previous
import math

import jax
import jax.numpy as jnp
from jax.experimental import pallas as pl
from jax.experimental.pallas import tpu as pltpu

B, N, F, H, T = 8, 512, 128, 128, 16
MASKED = -1.0
_INV_SQRT_H = 1.0 / math.sqrt(H)


def _edge_kernel(ns_ref, skip_ref, x_ref, adj_ref, a_ref,
                 we2h, be2h, wm_pr, wu_pr,
                 wm_bf, wu_bf, we12,
                 out_ref, zc_ref):
    g = pl.program_id(0)
    f32 = jnp.float32
    ns0 = ns_ref[2 * g]
    ns1 = ns_ref[2 * g + 1]
    nmin = jnp.minimum(ns0, ns1)
    adjs = (adj_ref[0], adj_ref[1])

    def net_step(l, z, h, wm, wu):
        m = jnp.maximum(jnp.dot(z + h, wm[...],
                                preferred_element_type=f32), 0.0)
        msg = jnp.dot(adjs[l], m, preferred_element_type=f32)
        zc_ref[l, :, H:2 * H] = msg
        return jnp.maximum(jnp.dot(zc_ref[l], wu[...],
                                   preferred_element_type=f32), 0.0)

    xs = x_ref[...].reshape(2 * N, F)
    z2s = jnp.tanh(jnp.dot(xs, we2h[...], preferred_element_type=f32)
                   + be2h[...])
    z2 = (z2s[0:N], z2s[N:2 * N])
    z_pr = (z2[0][:, 0:H], z2[1][:, 0:H])
    z_bf = (z2[0][:, H:2 * H], z2[1][:, H:2 * H])

    zc_ref[0, :, 0:H] = z_pr[0]
    zc_ref[1, :, 0:H] = z_pr[1]
    zero = jnp.zeros((N, H), f32)

    def joint(hh):
        return (net_step(0, z_pr[0], hh[0], wm_pr, wu_pr),
                net_step(1, z_pr[1], hh[1], wm_pr, wu_pr))

    hs = jax.lax.fori_loop(
        0, nmin // 2, lambda i, hh: joint(joint(hh)), (zero, zero))
    hs = jax.lax.fori_loop(
        2 * (nmin // 2), nmin, lambda i, hh: joint(hh), hs)

    h0 = jax.lax.fori_loop(
        nmin, ns0, lambda i, hh: net_step(0, z_pr[0], hh, wm_pr, wu_pr),
        hs[0])
    h1 = jax.lax.fori_loop(
        nmin, ns1, lambda i, hh: net_step(1, z_pr[1], hh, wm_pr, wu_pr),
        hs[1])

    zc_ref[0, :, 0:H] = z_bf[0]
    zc_ref[1, :, 0:H] = z_bf[1]
    for l, h in ((0, h0), (1, h1)):
        hb = net_step(l, z_bf[l], h, wm_bf, wu_bf)
        e12 = jnp.dot(hb, we12[...], preferred_element_type=f32)
        cand = jax.lax.dot_general(
            e12[:, 0:H], e12[:, H:2 * H], (((1,), (1,)), ((), ())),
            preferred_element_type=f32) * _INV_SQRT_H
        out_ref[l] = jnp.where(skip_ref[2 * g + l] != 0, MASKED,
                               a_ref[l] * cand)


def kernel(x, adj, A, W_enc_pr, b_enc_pr, W_msg_pr, W_upd_pr, w_node_pr,
           We1_pr, We2_pr, W_enc_bfs, b_enc_bfs, W_msg_bfs, W_upd_bfs,
           w_node_bfs, We1_bfs, We2_bfs, phase_logits, lengths):
    del w_node_pr, We1_pr, We2_pr, w_node_bfs

    p = jnp.argmax(phase_logits, axis=-1).astype(jnp.int32)
    iv = jnp.arange(T, dtype=jnp.int32)[:, None]
    ln = lengths.astype(jnp.int32)[None, :]
    valid = (iv >= 1) & (ln > iv + 1) & (p == 0)
    i_star = jnp.max(jnp.where(valid, iv, 0), axis=0)
    reset = (p == 1) & (iv < i_star[None, :])
    j0 = jnp.max(jnp.where(reset, iv + 1, 0), axis=0)
    nsteps = i_star - j0 + 1
    skip = ((i_star == 0) & (p[0] == 1)).astype(jnp.int32)

    we2h = jnp.concatenate([W_enc_pr, W_enc_bfs], axis=1)
    be2h = jnp.concatenate([b_enc_pr, b_enc_bfs]).reshape(1, 2 * H)
    we12 = jnp.concatenate([We1_bfs, We2_bfs], axis=1)

    smem = pl.BlockSpec(memory_space=pltpu.SMEM)
    full = lambda *shape: pl.BlockSpec(shape, lambda b: (0,) * len(shape))
    batched = lambda *shape: pl.BlockSpec((2,) + shape, lambda b: (b, 0, 0))

    out = pl.pallas_call(
        _edge_kernel,
        grid=(B // 2,),
        in_specs=[
            smem, smem,
            batched(N, F), batched(N, N), batched(N, N),
            full(F, 2 * H), full(1, 2 * H), full(H, H), full(2 * H, H),
            full(H, H), full(2 * H, H), full(H, 2 * H),
        ],
        out_specs=batched(N, N),
        out_shape=jax.ShapeDtypeStruct((B, N, N), jnp.float32),
        scratch_shapes=[pltpu.VMEM((2, N, 2 * H), jnp.float32)],
    )(nsteps, skip, x, adj, A,
      we2h, be2h, W_msg_pr, W_upd_pr, W_msg_bfs, W_upd_bfs, we12)
    return out
